# baseline (speedup 1.0000x reference)
import functools

import jax
import jax.numpy as jnp
from jax import lax
from jax.experimental import pallas as pl
from jax.experimental.pallas import tpu as pltpu
from jax.experimental.pallas import tpu_sc as plsc

E = 16384
D = 64
NUM_C = 5
N_ROWS = 100000
_NC = 2
_NS = 16
_NW = _NC * _NS
_CHUNK = 128
_ROWS_PER_W = E // _NW
_CHUNKS_PER_W = _ROWS_PER_W // _CHUNK
_BR = 20480


def _repack_body(xt_ref, o_ref):
    xt = xt_ref[...]
    ri = lax.broadcasted_iota(jnp.int32, (D, D), 0)
    ci = lax.broadcasted_iota(jnp.int32, (D, D), 1)
    eye = (ri == ci).astype(jnp.float32)
    x = lax.dot_general(xt, eye, (((0,), (0,)), ((), ())),
                        preferred_element_type=jnp.float32)
    o_ref[:, :D] = x
    o_ref[:, D:] = x


def _repack(tab_t, interpret=False):
    nb = (N_ROWS + _BR - 1) // _BR
    return pl.pallas_call(
        _repack_body,
        grid=(nb,),
        in_specs=[pl.BlockSpec((D, _BR), lambda i: (0, i))],
        out_specs=pl.BlockSpec((_BR, 2 * D), lambda i: (i, 0)),
        out_shape=jax.ShapeDtypeStruct((N_ROWS, 2 * D), jnp.float32),
        compiler_params=pltpu.CompilerParams(
            fuse_transposed_lhs_in_matmul=True),
        interpret=interpret,
    )(tab_t)


def _gather_body(tab, idx, out, idx2, rows, packed, sem):
    wid = lax.axis_index("s") * _NC + lax.axis_index("c")
    rbase = wid * _CHUNKS_PER_W
    pbase = wid * (_ROWS_PER_W // 2)
    pltpu.sync_copy(idx.at[pl.ds(rbase, _CHUNKS_PER_W)], idx2)
    copies = []
    for j in range(_CHUNKS_PER_W):
        sl = pl.ds(j * _CHUNK, _CHUNK)
        copies.append(pltpu.async_copy(tab.at[idx2.at[j]], rows.at[sl], sem))
    for c in copies:
        c.wait()

    def pack_row(j, carry):
        for k in range(4):
            sl = pl.ds(16 * k, 16)
            packed[j, sl] = rows[2 * j, sl]
        for k in range(4):
            sl = pl.ds(D + 16 * k, 16)
            packed[j, sl] = rows[2 * j + 1, sl]
        return carry

    lax.fori_loop(0, _ROWS_PER_W // 2, pack_row, 0)
    pltpu.sync_copy(packed, out.at[pl.ds(pbase, _ROWS_PER_W // 2)])


@functools.cache
def _sc_gather():
    return pl.kernel(
        _gather_body,
        out_type=jax.ShapeDtypeStruct((E // 2, 2 * D), jnp.float32),
        mesh=plsc.VectorSubcoreMesh(core_axis_name="c", subcore_axis_name="s"),
        scratch_types=(
            pltpu.VMEM((_CHUNKS_PER_W, _CHUNK), jnp.int32),
            pltpu.VMEM((_ROWS_PER_W, 2 * D), jnp.float32),
            pltpu.VMEM((_ROWS_PER_W // 2, 2 * D), jnp.float32),
            pltpu.SemaphoreType.DMA,
        ),
        compiler_params=pltpu.CompilerParams(use_tc_tiling_on_sc=True),
    )


def _softmax_rows(logits):
    m = jnp.max(logits, axis=1, keepdims=True)
    ex = jnp.exp(logits - m)
    return ex / jnp.sum(ex, axis=1, keepdims=True)


def _compute_body(u_ref, v_ref, w0_ref, w1_ref, w2_ref, ws_ref, out_ref):
    ue = u_ref[:, :D]
    uo = u_ref[:, D:]
    ve = v_ref[:, :D]
    vo = v_ref[:, D:]
    ws = ws_ref[...]
    le = lo = None
    for k, w_ref in enumerate((w0_ref, w1_ref, w2_ref)):
        w = w_ref[...]
        pe = jnp.dot(ue, w, preferred_element_type=jnp.float32)
        po = jnp.dot(uo, w, preferred_element_type=jnp.float32)
        xe = jnp.sum(pe * ve, axis=1, keepdims=True)
        xo = jnp.sum(po * vo, axis=1, keepdims=True)
        wsk = ws[k:k + 1, :]
        le = xe * wsk if le is None else le + xe * wsk
        lo = xo * wsk if lo is None else lo + xo * wsk
    out_ref[:, :NUM_C] = _softmax_rows(le)
    out_ref[:, NUM_C:] = _softmax_rows(lo)


def _tc_compute(u_g2, v_g2, W0, W1, W2, weights_scalars, block_e=4096,
                interpret=False):
    m_rows = E // 2
    grid = (m_rows // block_e,)
    return pl.pallas_call(
        _compute_body,
        grid=grid,
        in_specs=[
            pl.BlockSpec((block_e, 2 * D), lambda i: (i, 0)),
            pl.BlockSpec((block_e, 2 * D), lambda i: (i, 0)),
            pl.BlockSpec((D, D), lambda i: (0, 0)),
            pl.BlockSpec((D, D), lambda i: (0, 0)),
            pl.BlockSpec((D, D), lambda i: (0, 0)),
            pl.BlockSpec((3, NUM_C), lambda i: (0, 0)),
        ],
        out_specs=pl.BlockSpec((block_e, 2 * NUM_C), lambda i: (i, 0)),
        out_shape=jax.ShapeDtypeStruct((m_rows, 2 * NUM_C), jnp.float32),
        interpret=interpret,
    )(u_g2, v_g2, W0, W1, W2, weights_scalars)


def kernel(u_features, v_features, u_indices, v_indices, W0, W1, W2,
           weights_scalars):
    u_idx2 = u_indices.reshape(E // _CHUNK, _CHUNK)
    v_idx2 = v_indices.reshape(E // _CHUNK, _CHUNK)
    u_tab2 = _repack(u_features.T)
    u_g2 = _sc_gather()(u_tab2, u_idx2)
    v_tab2 = _repack(v_features.T)
    v_g2 = _sc_gather()(v_tab2, v_idx2)
    out10 = _tc_compute(u_g2, v_g2, W0, W1, W2, weights_scalars)
    return out10.reshape(E, NUM_C)


# speedup vs baseline: 1.1304x; 1.1304x over previous
import functools

import jax
import jax.numpy as jnp
from jax import lax
from jax.experimental import pallas as pl
from jax.experimental.pallas import tpu as pltpu
from jax.experimental.pallas import tpu_sc as plsc

E = 16384
D = 64
NUM_C = 5
N_ROWS = 100000
_NC = 2
_NS = 16
_NW = _NC * _NS
_CHUNK = 128
_ROWS_PER_W = E // _NW
_CHUNKS_PER_W = _ROWS_PER_W // _CHUNK
_BR = 20480


def _repack_body(xt_ref, o_ref):
    xt = xt_ref[...]
    ri = lax.broadcasted_iota(jnp.int32, (D, D), 0)
    ci = lax.broadcasted_iota(jnp.int32, (D, D), 1)
    eye = (ri == ci).astype(jnp.float32)
    x = lax.dot_general(xt, eye, (((0,), (0,)), ((), ())),
                        preferred_element_type=jnp.float32)
    o_ref[:, :D] = x
    o_ref[:, D:] = x


def _repack(tab_t, interpret=False):
    nb = (N_ROWS + _BR - 1) // _BR
    return pl.pallas_call(
        _repack_body,
        grid=(nb,),
        in_specs=[pl.BlockSpec((D, _BR), lambda i: (0, i))],
        out_specs=pl.BlockSpec((_BR, 2 * D), lambda i: (i, 0)),
        out_shape=jax.ShapeDtypeStruct((N_ROWS, 2 * D), jnp.float32),
        compiler_params=pltpu.CompilerParams(
            fuse_transposed_lhs_in_matmul=True),
        interpret=interpret,
    )(tab_t)


def _gather_body(tab, idx, out, idx2, rows, sem):
    wid = lax.axis_index("s") * _NC + lax.axis_index("c")
    rbase = wid * _CHUNKS_PER_W
    base = wid * _ROWS_PER_W
    pltpu.sync_copy(idx.at[pl.ds(rbase, _CHUNKS_PER_W)], idx2)
    copies = []
    for j in range(_CHUNKS_PER_W):
        sl = pl.ds(j * _CHUNK, _CHUNK)
        copies.append(pltpu.async_copy(tab.at[idx2.at[j]], rows.at[sl], sem))
    for c in copies:
        c.wait()
    pltpu.sync_copy(rows, out.at[pl.ds(base, _ROWS_PER_W)])


@functools.cache
def _sc_gather():
    return pl.kernel(
        _gather_body,
        out_type=jax.ShapeDtypeStruct((E, 2 * D), jnp.float32),
        mesh=plsc.VectorSubcoreMesh(core_axis_name="c", subcore_axis_name="s"),
        scratch_types=(
            pltpu.VMEM((_CHUNKS_PER_W, _CHUNK), jnp.int32),
            pltpu.VMEM((_ROWS_PER_W, 2 * D), jnp.float32),
            pltpu.SemaphoreType.DMA,
        ),
        compiler_params=pltpu.CompilerParams(use_tc_tiling_on_sc=True),
    )


def _softmax_rows(logits):
    m = jnp.max(logits, axis=1, keepdims=True)
    ex = jnp.exp(logits - m)
    return ex / jnp.sum(ex, axis=1, keepdims=True)


def _compute_body(u_ref, v_ref, w0_ref, w1_ref, w2_ref, ws_ref, out_ref):
    u = u_ref[:, :D]
    v = v_ref[:, :D]
    ws = ws_ref[...]
    logits = None
    for k, w_ref in enumerate((w0_ref, w1_ref, w2_ref)):
        p = jnp.dot(u, w_ref[...], preferred_element_type=jnp.float32)
        x = jnp.sum(p * v, axis=1, keepdims=True)
        contrib = x * ws[k:k + 1, :]
        logits = contrib if logits is None else logits + contrib
    out_ref[...] = _softmax_rows(logits)


def _tc_compute(u_g2, v_g2, W0, W1, W2, weights_scalars, block_e=8192,
                interpret=False):
    grid = (E // block_e,)
    return pl.pallas_call(
        _compute_body,
        grid=grid,
        in_specs=[
            pl.BlockSpec((block_e, 2 * D), lambda i: (i, 0)),
            pl.BlockSpec((block_e, 2 * D), lambda i: (i, 0)),
            pl.BlockSpec((D, D), lambda i: (0, 0)),
            pl.BlockSpec((D, D), lambda i: (0, 0)),
            pl.BlockSpec((D, D), lambda i: (0, 0)),
            pl.BlockSpec((3, NUM_C), lambda i: (0, 0)),
        ],
        out_specs=pl.BlockSpec((block_e, NUM_C), lambda i: (i, 0)),
        out_shape=jax.ShapeDtypeStruct((E, NUM_C), jnp.float32),
        interpret=interpret,
    )(u_g2, v_g2, W0, W1, W2, weights_scalars)


def kernel(u_features, v_features, u_indices, v_indices, W0, W1, W2,
           weights_scalars):
    u_idx2 = u_indices.reshape(E // _CHUNK, _CHUNK)
    v_idx2 = v_indices.reshape(E // _CHUNK, _CHUNK)
    u_tab2 = _repack(u_features.T)
    u_g2 = _sc_gather()(u_tab2, u_idx2)
    v_tab2 = _repack(v_features.T)
    v_g2 = _sc_gather()(v_tab2, v_idx2)
    return _tc_compute(u_g2, v_g2, W0, W1, W2, weights_scalars)
